# baseline (speedup 1.0000x reference)
import jax
import jax.numpy as jnp
from jax import lax
from jax.experimental import pallas as pl
from jax.experimental.pallas import tpu as pltpu
import jax.experimental.pallas.tpu_sc as plsc

N = 10000
E = 320000
D = 128
OUT = 128

NC = 2
NS = 16
K = 40
EPT = E // NS
NCH = 504
ETP = NCH * K
KI = 3
NGI = NCH // KI
NB = 3
NPAD = 10240
ROWS = NPAD // NS
TRASH = 10200


def _sc_agg_body(xp_hbm, epad_hbm, zrow_hbm, zdeg_hbm,
                 sum_out, deg_out,
                 gi_v, si_v, deg_v, acc_s, *bufs_and_sems):
    rows = list(bufs_and_sems[0:NB])
    sem_g = list(bufs_and_sems[NB:2 * NB])
    sem_s = list(bufs_and_sems[2 * NB:3 * NB])
    sem_ig = list(bufs_and_sems[3 * NB:3 * NB + 2])
    sem_is = list(bufs_and_sems[3 * NB + 2:3 * NB + 4])

    cid = lax.axis_index("c")
    sid = lax.axis_index("s")
    sl = pl.ds(sid * ROWS, ROWS)
    ones16 = jnp.ones((16,), jnp.float32)

    def issue_idx(g, slot):
        src = pl.ds(g * KI, KI)
        pltpu.async_copy(epad_hbm.at[cid, sid, src], gi_v.at[slot],
                         sem_ig[slot])
        pltpu.async_copy(epad_hbm.at[1 - cid, sid, src], si_v.at[slot],
                         sem_is[slot])

    def wait_idx(slot):
        dummy = epad_hbm.at[0, 0, pl.ds(0, KI)]
        pltpu.make_async_copy(dummy, gi_v.at[slot], sem_ig[slot]).wait()
        pltpu.make_async_copy(dummy, si_v.at[slot], sem_is[slot]).wait()

    def issue_gather(slot, row, b):
        pltpu.async_copy(xp_hbm.at[gi_v.at[slot, row]], rows[b], sem_g[b])

    def wait_gather(b):
        pltpu.make_async_copy(xp_hbm.at[pl.ds(0, K)], rows[b],
                              sem_g[b]).wait()

    def issue_scatter(slot, row, b):
        pltpu.async_copy(rows[b], acc_s.at[si_v.at[slot, row]],
                         sem_s[b], add=True)

    def wait_scatter(b):
        pltpu.make_async_copy(rows[b], acc_s.at[pl.ds(0, K)],
                              sem_s[b]).wait()

    def count_degrees(slot, row):
        for u in range(K // 16):
            idx = si_v[slot, row, pl.ds(u * 16, 16)]
            plsc.addupdate_scatter(deg_v, [idx], ones16)

    pltpu.sync_copy(zrow_hbm, acc_s.at[sl])
    pltpu.sync_copy(zdeg_hbm, deg_v)

    issue_idx(0, 0)
    wait_idx(0)
    plsc.subcore_barrier()
    for c in range(NB):
        issue_gather(0, c, c)

    def pair(gp, carry):
        for gg in range(2):
            g = 2 * gp + gg
            for q in range(KI):
                bp = (q - 1) % NB
                if gg == 0 and q == 0:
                    @pl.when(gp > 0)
                    def _():
                        wait_scatter(bp)
                        issue_gather(gg, 2, bp)
                    issue_idx(g + 1, 1 - gg)
                elif q == 0:
                    wait_scatter(bp)
                    issue_gather(gg, 2, bp)
                    issue_idx(g + 1, 1 - gg)
                elif q == 1:
                    wait_scatter(bp)
                    wait_idx(1 - gg)
                    issue_gather(1 - gg, 0, bp)
                else:
                    wait_scatter(bp)
                    issue_gather(1 - gg, 1, bp)
                wait_gather(q)
                issue_scatter(gg, q, q)
                count_degrees(gg, q)
        return carry

    lax.fori_loop(0, NGI // 2 - 1, pair, 0)

    for gg in range(2):
        for q in range(KI):
            bp = (q - 1) % NB
            wait_scatter(bp)
            if gg == 0:
                if q == 0:
                    issue_gather(0, 2, bp)
                    issue_idx(NGI - 1, 1)
                elif q == 1:
                    wait_idx(1)
                    issue_gather(1, 0, bp)
                else:
                    issue_gather(1, 1, bp)
            else:
                if q == 0:
                    issue_gather(1, 2, bp)
            wait_gather(q)
            issue_scatter(gg, q, q)
            count_degrees(gg, q)
    wait_scatter((NCH - 1) % NB)

    plsc.subcore_barrier()

    pltpu.sync_copy(acc_s.at[sl], sum_out.at[cid, sl])
    pltpu.sync_copy(deg_v, deg_out.at[cid, sid])


def _sc_aggregate(xp, epad, zrow, zdeg):
    mesh = plsc.VectorSubcoreMesh(core_axis_name="c", subcore_axis_name="s")
    return pl.kernel(
        _sc_agg_body,
        out_type=(
            jax.ShapeDtypeStruct((NC, NPAD, D), jnp.float32),
            jax.ShapeDtypeStruct((NC, NS, NPAD), jnp.float32),
        ),
        mesh=mesh,
        compiler_params=pltpu.CompilerParams(use_tc_tiling_on_sc=False,
                                             needs_layout_passes=False),
        scratch_types=(
            [pltpu.VMEM((2, KI, K), jnp.int32),
             pltpu.VMEM((2, KI, K), jnp.int32),
             pltpu.VMEM((NPAD,), jnp.float32),
             pltpu.VMEM_SHARED((NPAD, D), jnp.float32)]
            + [pltpu.VMEM((K, D), jnp.float32) for _ in range(NB)]
            + [pltpu.SemaphoreType.DMA for _ in range(2 * NB + 4)]
        ),
    )(xp, epad, zrow, zdeg)


def _linear_body(x_ref, fs_ref, bs_ref,
                 fd_ref, bd_ref, w_ref, b_ref, o_ref):
    x = x_ref[...]
    fd = jnp.sum(fd_ref[0], axis=0)[:, None]
    bd = jnp.sum(bd_ref[0], axis=0)[:, None]
    fr = 1.0 / jnp.maximum(fd, 1.0)
    br = 1.0 / jnp.maximum(bd, 1.0)
    h = jnp.concatenate([x, fs_ref[0] * fr, bs_ref[0] * br], axis=1)
    o_ref[...] = lax.dot_general(
        h, w_ref[...], (((1,), (1,)), ((), ())),
        preferred_element_type=jnp.float32) + b_ref[...]


def _linear(x, sums, degs, W, b2):
    R = 1024
    grid = (NPAD // R,)
    return pl.pallas_call(
        _linear_body,
        grid=grid,
        in_specs=[
            pl.BlockSpec((R, D), lambda i: (i, 0)),
            pl.BlockSpec((1, R, D), lambda i: (0, i, 0)),
            pl.BlockSpec((1, R, D), lambda i: (1, i, 0)),
            pl.BlockSpec((1, NS, R), lambda i: (0, 0, i)),
            pl.BlockSpec((1, NS, R), lambda i: (1, 0, i)),
            pl.BlockSpec((OUT, 3 * D), lambda i: (0, 0)),
            pl.BlockSpec((1, OUT), lambda i: (0, 0)),
        ],
        out_specs=pl.BlockSpec((R, OUT), lambda i: (i, 0)),
        out_shape=jax.ShapeDtypeStruct((N, OUT), jnp.float32),
    )(x, sums, sums, degs, degs, W, b2)


@jax.jit
def kernel(x, edge_index, W, b):
    xp = jnp.pad(x, ((0, NPAD - N), (0, 0)))
    ei3 = edge_index.reshape(2, NS, EPT)
    pad = jnp.full((2, NS, ETP - EPT), TRASH, jnp.int32)
    epad = jnp.concatenate([ei3, pad], axis=2).reshape(2, NS, NCH, K)
    zrow = jnp.zeros((ROWS, D), jnp.float32)
    zdeg = jnp.zeros((NPAD,), jnp.float32)
    sums, degs = _sc_aggregate(xp, epad, zrow, zdeg)
    return _linear(x, sums, degs, W, b.reshape(1, OUT))


# speedup vs baseline: 2.0614x; 2.0614x over previous
import jax
import jax.numpy as jnp
from jax import lax
from jax.experimental import pallas as pl
from jax.experimental.pallas import tpu as pltpu
import jax.experimental.pallas.tpu_sc as plsc

N = 10000
E = 320000
D = 128
OUT = 128

NC = 2
NS = 16
K = 80
H = D // 2
EPT = E // NS
NCHUNK = EPT // K
NB = 5
NGRP = NCHUNK // NB
NPAD = 10240
ROWS = NPAD // NS


def _sc_agg_body(xv_hbm, eidx_hbm, zrow_hbm, zdeg_hbm,
                 sum_out, deg_out,
                 gidx_v, sidx_v, deg_v, acc_s, *bufs_and_sems):
    rows = list(bufs_and_sems[0:NB])
    sem_g = list(bufs_and_sems[NB:2 * NB])
    sem_s = list(bufs_and_sems[2 * NB:3 * NB])

    cid = lax.axis_index("c")
    sid = lax.axis_index("s")
    sl = pl.ds(sid * ROWS, ROWS)
    ones16 = jnp.ones((16,), jnp.float32)

    pltpu.sync_copy(eidx_hbm.at[cid, pl.ds(sid * EPT, EPT)], gidx_v)
    pltpu.sync_copy(eidx_hbm.at[1 - cid, pl.ds(sid * EPT, EPT)], sidx_v)

    pltpu.sync_copy(zrow_hbm, acc_s.at[sl])
    pltpu.sync_copy(zdeg_hbm, deg_v)

    def dbl(k, carry):
        v = gidx_v[pl.ds(k * 16, 16)]
        gidx_v[pl.ds(k * 16, 16)] = v + v
        return carry

    lax.fori_loop(0, EPT // 16, dbl, 0)

    def wait_gather(b):
        pltpu.make_async_copy(xv_hbm.at[pl.ds(0, K)], rows[b],
                              sem_g[b]).wait()

    def issue_scatter(jj, b):
        pltpu.async_copy(rows[b], acc_s.at[sidx_v.at[pl.ds(jj * K, K)]],
                         sem_s[b], add=True)

    def wait_scatter(b):
        pltpu.make_async_copy(rows[b], acc_s.at[pl.ds(0, K)],
                              sem_s[b]).wait()

    def count_degrees(jj):
        for u in range(K // 16):
            idx = sidx_v[pl.ds(jj * K + u * 16, 16)]
            plsc.addupdate_scatter(deg_v, [idx], ones16)

    for p in range(2):
        tbl = xv_hbm if p == 0 else xv_hbm.at[pl.ds(1, 2 * N - 1)]

        def issue_gather(jj, b, tbl=tbl):
            pltpu.async_copy(tbl.at[gidx_v.at[pl.ds(jj * K, K)]],
                             rows[b], sem_g[b])

        plsc.subcore_barrier()

        for b in range(NB):
            issue_gather(b, b)

        def group(g, carry):
            for b in range(NB):
                jj = g * NB + b
                bp = (b - 1) % NB
                if b == 0:
                    @pl.when(g > 0)
                    def _():
                        wait_scatter(bp)
                        issue_gather(jj - 1 + NB, bp)
                else:
                    wait_scatter(bp)
                    issue_gather(jj - 1 + NB, bp)
                wait_gather(b)
                issue_scatter(jj, b)
                if p == 0:
                    count_degrees(jj)
            return carry

        lax.fori_loop(0, NGRP - 1, group, 0)

        for b in range(NB):
            jj = (NGRP - 1) * NB + b
            bp = (b - 1) % NB
            wait_scatter(bp)
            if b == 0:
                issue_gather(jj - 1 + NB, bp)
            wait_gather(b)
            issue_scatter(jj, b)
            if p == 0:
                count_degrees(jj)
        wait_scatter((NCHUNK - 1) % NB)

        plsc.subcore_barrier()

        pltpu.sync_copy(acc_s.at[sl],
                        sum_out.at[cid, sl, pl.ds(p * H, H)])
        if p == 0:
            pltpu.sync_copy(deg_v, deg_out.at[cid, sid])
            pltpu.sync_copy(zrow_hbm, acc_s.at[sl])


def _sc_aggregate(xv, eidx, zrow, zdeg):
    mesh = plsc.VectorSubcoreMesh(core_axis_name="c", subcore_axis_name="s")
    return pl.kernel(
        _sc_agg_body,
        out_type=(
            jax.ShapeDtypeStruct((NC, NPAD, D), jnp.float32),
            jax.ShapeDtypeStruct((NC, NS, NPAD), jnp.float32),
        ),
        mesh=mesh,
        compiler_params=pltpu.CompilerParams(use_tc_tiling_on_sc=False,
                                             needs_layout_passes=False),
        scratch_types=(
            [pltpu.VMEM((EPT,), jnp.int32),
             pltpu.VMEM((EPT,), jnp.int32),
             pltpu.VMEM((NPAD,), jnp.float32),
             pltpu.VMEM_SHARED((NPAD, H), jnp.float32)]
            + [pltpu.VMEM((K, H), jnp.float32) for _ in range(NB)]
            + [pltpu.SemaphoreType.DMA for _ in range(2 * NB)]
        ),
    )(xv, eidx, zrow, zdeg)


def _linear_body(x_ref, fs_ref, bs_ref,
                 fd_ref, bd_ref, w_ref, b_ref, o_ref):
    x = x_ref[...]
    fd = jnp.sum(fd_ref[0], axis=0)[:, None]
    bd = jnp.sum(bd_ref[0], axis=0)[:, None]
    fr = 1.0 / jnp.maximum(fd, 1.0)
    br = 1.0 / jnp.maximum(bd, 1.0)
    h = jnp.concatenate([x, fs_ref[0] * fr, bs_ref[0] * br], axis=1)
    o_ref[...] = lax.dot_general(
        h, w_ref[...], (((1,), (1,)), ((), ())),
        preferred_element_type=jnp.float32) + b_ref[...]


def _linear(x, sums, degs, W, b2):
    R = 1024
    grid = (NPAD // R,)
    return pl.pallas_call(
        _linear_body,
        grid=grid,
        in_specs=[
            pl.BlockSpec((R, D), lambda i: (i, 0)),
            pl.BlockSpec((1, R, D), lambda i: (0, i, 0)),
            pl.BlockSpec((1, R, D), lambda i: (1, i, 0)),
            pl.BlockSpec((1, NS, R), lambda i: (0, 0, i)),
            pl.BlockSpec((1, NS, R), lambda i: (1, 0, i)),
            pl.BlockSpec((OUT, 3 * D), lambda i: (0, 0)),
            pl.BlockSpec((1, OUT), lambda i: (0, 0)),
        ],
        out_specs=pl.BlockSpec((R, OUT), lambda i: (i, 0)),
        out_shape=jax.ShapeDtypeStruct((N, OUT), jnp.float32),
    )(x, sums, sums, degs, degs, W, b2)


@jax.jit
def kernel(x, edge_index, W, b):
    xv = x.reshape(2 * N, H)
    zrow = jnp.zeros((ROWS, H), jnp.float32)
    zdeg = jnp.zeros((NPAD,), jnp.float32)
    sums, degs = _sc_aggregate(xv, edge_index, zrow, zdeg)
    return _linear(x, sums, degs, W, b.reshape(1, OUT))
